# baseline (speedup 1.0000x reference)
import functools

import jax
import jax.numpy as jnp
from jax import lax
from jax.experimental import pallas as pl
from jax.experimental.pallas import tpu as pltpu
from jax.experimental.pallas import tpu_sc as plsc

_K = 8192
_KBITS = 13
_DC = 32
_D = 1024
_BL = 4 * 8192

_NC = 2
_NS = 16
_NW = _NC * _NS
_TPW = _BL // _NW
_CHUNK = 128
_NCHUNK = _TPW // _CHUNK

_TBLK = 1024
_NBLK = _BL // _TBLK


def _sc_gather_body(table_hbm, idx_hbm, y0_hbm, y1_hbm,
                    idx_v, d0_v, d1_v, rows0_v, rows1_v, sem):
    c = lax.axis_index("c")
    s = lax.axis_index("s")
    wid = s * _NC + c
    pltpu.sync_copy(idx_hbm.at[pl.ds(wid * _NCHUNK, _NCHUNK)], idx_v)
    for j in range(_NCHUNK):
        for t in range(_CHUNK // 16):
            v = idx_v[j, pl.ds(t * 16, 16)]
            d0_v[j, pl.ds(t * 16, 16)] = v & (_K - 1)
            d1_v[j, pl.ds(t * 16, 16)] = (v >> _KBITS) + _K
    copies = []
    for j in range(_NCHUNK):
        copies.append(pltpu.async_copy(
            table_hbm.at[d0_v.at[j]],
            rows0_v.at[pl.ds(j * _CHUNK, _CHUNK)], sem))
        copies.append(pltpu.async_copy(
            table_hbm.at[d1_v.at[j]],
            rows1_v.at[pl.ds(j * _CHUNK, _CHUNK)], sem))
    for cp in copies:
        cp.wait()
    base = wid * _TPW
    pltpu.sync_copy(rows0_v, y0_hbm.at[pl.ds(base, _TPW)])
    pltpu.sync_copy(rows1_v, y1_hbm.at[pl.ds(base, _TPW)])


@functools.cache
def _make_sc_gather():
    return functools.partial(
        pl.kernel,
        out_type=(
            jax.ShapeDtypeStruct((_BL, _DC), jnp.float32),
            jax.ShapeDtypeStruct((_BL, _DC), jnp.float32),
        ),
        mesh=plsc.VectorSubcoreMesh(core_axis_name="c", subcore_axis_name="s",
                                    num_cores=_NC, num_subcores=_NS),
        scratch_types=[
            pltpu.VMEM((_NCHUNK, _CHUNK), jnp.int32),
            pltpu.VMEM((_NCHUNK, _CHUNK), jnp.int32),
            pltpu.VMEM((_NCHUNK, _CHUNK), jnp.int32),
            pltpu.VMEM((_TPW, _DC), jnp.float32),
            pltpu.VMEM((_TPW, _DC), jnp.float32),
            pltpu.SemaphoreType.DMA,
        ],
        compiler_params=pltpu.CompilerParams(use_tc_tiling_on_sc=False),
    )(_sc_gather_body)


def _tc_body(y0_ref, y1_ref, idx_ref, wdc_ref, spe_ref, out_ref):
    y = y0_ref[...] + y1_ref[...]
    z = jnp.dot(y, wdc_ref[...], preferred_element_type=jnp.float32)
    iv = idx_ref[0]
    mask = iv < 4
    eq = iv == lax.broadcasted_iota(jnp.int32, (_TBLK, 8), 1)
    oh = jnp.where(eq & mask, 1.0, 0.0)
    sp = jnp.dot(oh, spe_ref[...], preferred_element_type=jnp.float32)
    out_ref[...] = jnp.where(mask, sp, z)


def kernel(idx, codebooks, Wdc, special_emb):
    idx = idx.astype(jnp.int32)
    table = codebooks.reshape(2 * _K, _DC)
    idx_flat = idx.reshape(-1)
    y0, y1 = _make_sc_gather()(table, idx_flat.reshape(_NW * _NCHUNK, _CHUNK))
    spe = jnp.concatenate(
        [special_emb, jnp.zeros((8 - special_emb.shape[0], _D),
                                dtype=special_emb.dtype)], axis=0)
    out = pl.pallas_call(
        _tc_body,
        grid=(_NBLK,),
        in_specs=[
            pl.BlockSpec((_TBLK, _DC), lambda i: (i, 0)),
            pl.BlockSpec((_TBLK, _DC), lambda i: (i, 0)),
            pl.BlockSpec((1, _TBLK, 1), lambda i: (i, 0, 0)),
            pl.BlockSpec((_DC, _D), lambda i: (0, 0)),
            pl.BlockSpec((8, _D), lambda i: (0, 0)),
        ],
        out_specs=pl.BlockSpec((_TBLK, _D), lambda i: (i, 0)),
        out_shape=jax.ShapeDtypeStruct((_BL, _D), jnp.float32),
    )(y0, y1, idx_flat.reshape(_NBLK, _TBLK, 1), Wdc, spe)
    return out.reshape(idx.shape + (_D,))


# speedup vs baseline: 1.1598x; 1.1598x over previous
import functools

import jax
import jax.numpy as jnp
from jax import lax
from jax.experimental import pallas as pl
from jax.experimental.pallas import tpu as pltpu
from jax.experimental.pallas import tpu_sc as plsc

_K = 8192
_KBITS = 13
_DC = 32
_D = 1024
_BL = 4 * 8192

_NC = 2
_NS = 16
_NW = _NC * _NS
_TPW = _BL // _NW
_CHUNK = 128
_NCHUNK = _TPW // _CHUNK

_TBLK = 2048
_NBLK = _BL // _TBLK


def _sc_gather_body(table_hbm, idx_hbm, y_hbm,
                    idx_v, d0_v, d1_v, rows0_v, rows1_v, gsem, ssem):
    c = lax.axis_index("c")
    s = lax.axis_index("s")
    wid = s * _NC + c
    pltpu.sync_copy(idx_hbm.at[pl.ds(wid * _NCHUNK, _NCHUNK)], idx_v)
    for j in range(_NCHUNK):
        for t in range(_CHUNK // 16):
            v = idx_v[j, pl.ds(t * 16, 16)]
            d0_v[j, pl.ds(t * 16, 16)] = v & (_K - 1)
            d1_v[j, pl.ds(t * 16, 16)] = (v >> _KBITS) + _K
    copies = []
    for j in range(_NCHUNK):
        copies.append(pltpu.async_copy(
            table_hbm.at[d0_v.at[j]],
            rows0_v.at[pl.ds(j * _CHUNK, _CHUNK)], gsem))
        copies.append(pltpu.async_copy(
            table_hbm.at[d1_v.at[j]],
            rows1_v.at[pl.ds(j * _CHUNK, _CHUNK)], gsem))
    base = wid * _TPW

    def _add_row(r, carry):
        rows0_v[r, pl.ds(0, 16)] = rows0_v[r, pl.ds(0, 16)] + rows1_v[r, pl.ds(0, 16)]
        rows0_v[r, pl.ds(16, 16)] = rows0_v[r, pl.ds(16, 16)] + rows1_v[r, pl.ds(16, 16)]
        return carry

    outs = []
    for j in range(_NCHUNK):
        copies[2 * j].wait()
        copies[2 * j + 1].wait()
        lax.fori_loop(j * _CHUNK, (j + 1) * _CHUNK, _add_row, 0)
        outs.append(pltpu.async_copy(
            rows0_v.at[pl.ds(j * _CHUNK, _CHUNK)],
            y_hbm.at[pl.ds(base + j * _CHUNK, _CHUNK)], ssem))
    for cp in outs:
        cp.wait()


@functools.cache
def _make_sc_gather():
    return functools.partial(
        pl.kernel,
        out_type=jax.ShapeDtypeStruct((_BL, _DC), jnp.float32),
        mesh=plsc.VectorSubcoreMesh(core_axis_name="c", subcore_axis_name="s",
                                    num_cores=_NC, num_subcores=_NS),
        scratch_types=[
            pltpu.VMEM((_NCHUNK, _CHUNK), jnp.int32),
            pltpu.VMEM((_NCHUNK, _CHUNK), jnp.int32),
            pltpu.VMEM((_NCHUNK, _CHUNK), jnp.int32),
            pltpu.VMEM((_TPW, _DC), jnp.float32),
            pltpu.VMEM((_TPW, _DC), jnp.float32),
            pltpu.SemaphoreType.DMA,
            pltpu.SemaphoreType.DMA,
        ],
        compiler_params=pltpu.CompilerParams(use_tc_tiling_on_sc=False),
    )(_sc_gather_body)


def _tc_body(y_ref, idx_ref, wdc_ref, spe_ref, out_ref):
    z = jnp.dot(y_ref[...], wdc_ref[...], preferred_element_type=jnp.float32)
    iv = idx_ref[0]
    mask = iv < 4
    eq = iv == lax.broadcasted_iota(jnp.int32, (_TBLK, 8), 1)
    oh = jnp.where(eq & mask, 1.0, 0.0)
    sp = jnp.dot(oh, spe_ref[...], preferred_element_type=jnp.float32)
    out_ref[...] = jnp.where(mask, sp, z)


def kernel(idx, codebooks, Wdc, special_emb):
    idx = idx.astype(jnp.int32)
    table = codebooks.reshape(2 * _K, _DC)
    idx_flat = idx.reshape(-1)
    y = _make_sc_gather()(table, idx_flat.reshape(_NW * _NCHUNK, _CHUNK))
    spe = jnp.concatenate(
        [special_emb, jnp.zeros((8 - special_emb.shape[0], _D),
                                dtype=special_emb.dtype)], axis=0)
    out = pl.pallas_call(
        _tc_body,
        grid=(_NBLK,),
        in_specs=[
            pl.BlockSpec((_TBLK, _DC), lambda i: (i, 0)),
            pl.BlockSpec((1, _TBLK, 1), lambda i: (i, 0, 0)),
            pl.BlockSpec((_DC, _D), lambda i: (0, 0)),
            pl.BlockSpec((8, _D), lambda i: (0, 0)),
        ],
        out_specs=pl.BlockSpec((_TBLK, _D), lambda i: (i, 0)),
        out_shape=jax.ShapeDtypeStruct((_BL, _D), jnp.float32),
    )(y, idx_flat.reshape(_NBLK, _TBLK, 1), Wdc, spe)
    return out.reshape(idx.shape + (_D,))


# speedup vs baseline: 1.1744x; 1.0126x over previous
import functools

import jax
import jax.numpy as jnp
from jax import lax
from jax.experimental import pallas as pl
from jax.experimental.pallas import tpu as pltpu
from jax.experimental.pallas import tpu_sc as plsc

_K = 8192
_KBITS = 13
_DC = 32
_D = 1024
_BL = 4 * 8192

_NC = 2
_NS = 16
_NW = _NC * _NS
_TPW = _BL // _NW
_CHUNK = 128
_NCHUNK = _TPW // _CHUNK

_TBLK = 2048
_NBLK = _BL // _TBLK


def _sc_gather_body(table_hbm, idx_hbm, y_hbm,
                    idx_v, d0_v, d1_v, rows0_v, rows1_v, gsem, ssem):
    c = lax.axis_index("c")
    s = lax.axis_index("s")
    wid = s * _NC + c
    pltpu.sync_copy(idx_hbm.at[pl.ds(wid * _NCHUNK, _NCHUNK)], idx_v)
    for j in range(_NCHUNK):
        for t in range(_CHUNK // 16):
            v = idx_v[j, pl.ds(t * 16, 16)]
            d0_v[j, pl.ds(t * 16, 16)] = v & (_K - 1)
            d1_v[j, pl.ds(t * 16, 16)] = (v >> _KBITS) + _K
    copies = []
    for j in range(_NCHUNK):
        copies.append(pltpu.async_copy(
            table_hbm.at[d0_v.at[j]],
            rows0_v.at[pl.ds(j * _CHUNK, _CHUNK)], gsem))
        copies.append(pltpu.async_copy(
            table_hbm.at[d1_v.at[j]],
            rows1_v.at[pl.ds(j * _CHUNK, _CHUNK)], gsem))
    base = wid * _TPW

    def _add_row(r, carry):
        rows0_v[r, pl.ds(0, 16)] = rows0_v[r, pl.ds(0, 16)] + rows1_v[r, pl.ds(0, 16)]
        rows0_v[r, pl.ds(16, 16)] = rows0_v[r, pl.ds(16, 16)] + rows1_v[r, pl.ds(16, 16)]
        return carry

    outs = []
    for j in range(_NCHUNK):
        copies[2 * j].wait()
        copies[2 * j + 1].wait()
        lax.fori_loop(j * _CHUNK, (j + 1) * _CHUNK, _add_row, 0)
        outs.append(pltpu.async_copy(
            rows0_v.at[pl.ds(j * _CHUNK, _CHUNK)],
            y_hbm.at[pl.ds(base + j * _CHUNK, _CHUNK)], ssem))
    for cp in outs:
        cp.wait()


@functools.cache
def _make_sc_gather():
    return functools.partial(
        pl.kernel,
        out_type=jax.ShapeDtypeStruct((_BL, _DC), jnp.float32),
        mesh=plsc.VectorSubcoreMesh(core_axis_name="c", subcore_axis_name="s",
                                    num_cores=_NC, num_subcores=_NS),
        scratch_types=[
            pltpu.VMEM((_NCHUNK, _CHUNK), jnp.int32),
            pltpu.VMEM((_NCHUNK, _CHUNK), jnp.int32),
            pltpu.VMEM((_NCHUNK, _CHUNK), jnp.int32),
            pltpu.VMEM((_TPW, _DC), jnp.float32),
            pltpu.VMEM((_TPW, _DC), jnp.float32),
            pltpu.SemaphoreType.DMA,
            pltpu.SemaphoreType.DMA,
        ],
        compiler_params=pltpu.CompilerParams(use_tc_tiling_on_sc=False),
    )(_sc_gather_body)


def _tc_body(y_ref, idx_ref, w_ref, out_ref):
    iv = idx_ref[0]
    mask = iv < 4
    y = jnp.where(mask, 0.0, y_ref[...])
    eq = iv == lax.broadcasted_iota(jnp.int32, (_TBLK, 8), 1)
    oh = jnp.where(eq & mask, 1.0, 0.0)
    ycat = jnp.concatenate([y, oh], axis=1)
    out_ref[...] = jnp.dot(ycat, w_ref[...], preferred_element_type=jnp.float32)


def kernel(idx, codebooks, Wdc, special_emb):
    idx = idx.astype(jnp.int32)
    table = codebooks.reshape(2 * _K, _DC)
    idx_flat = idx.reshape(-1)
    y = _make_sc_gather()(table, idx_flat.reshape(_NW * _NCHUNK, _CHUNK))
    wcat = jnp.concatenate(
        [Wdc, special_emb,
         jnp.zeros((4, _D), dtype=Wdc.dtype)], axis=0)
    out = pl.pallas_call(
        _tc_body,
        grid=(_NBLK,),
        in_specs=[
            pl.BlockSpec((_TBLK, _DC), lambda i: (i, 0)),
            pl.BlockSpec((1, _TBLK, 1), lambda i: (i, 0, 0)),
            pl.BlockSpec((_DC + 8, _D), lambda i: (0, 0)),
        ],
        out_specs=pl.BlockSpec((_TBLK, _D), lambda i: (i, 0)),
        out_shape=jax.ShapeDtypeStruct((_BL, _D), jnp.float32),
    )(y, idx_flat.reshape(_NBLK, _TBLK, 1), wcat)
    return out.reshape(idx.shape + (_D,))
